# baseline (speedup 1.0000x reference)
import functools

import jax
import jax.numpy as jnp
from jax.experimental import pallas as pl
from jax.experimental.pallas import tpu as pltpu

N = 10000
E = 320000
C = 64
NLAYERS = 5
NB = 2000
EB = 2000


def _leaky(v):
    return jnp.where(v > 0, v, 0.2 * v)


def _bdot(a, b):
    return jax.lax.dot(a.astype(jnp.bfloat16), b.astype(jnp.bfloat16),
                       preferred_element_type=jnp.float32)


def _node_first_body(x_ref, w_ref, asrc_ref, adst_ref, hw_ref, s_ref, d_ref):
    hw = _bdot(x_ref[...], w_ref[...])
    hw_ref[...] = hw
    s_ref[...] = (hw * asrc_ref[...]).sum(-1)[:, None]
    d_ref[...] = (hw * adst_ref[...]).sum(-1)[:, None]


def _node_body(acc_ref, den_ref, bias_ref, w_ref, asrc_ref, adst_ref,
               hw_ref, s_ref, d_ref):
    dn = den_ref[0, :, 0] + den_ref[1, :, 0]
    inv = (1.0 / (dn + 1e-16))[:, None]
    act = _leaky((acc_ref[0] + acc_ref[1]) * inv + bias_ref[...])
    hw = _bdot(act, w_ref[...])
    hw_ref[...] = hw
    s_ref[...] = (hw * asrc_ref[...]).sum(-1)[:, None]
    d_ref[...] = (hw * adst_ref[...]).sum(-1)[:, None]


def _rep(shape):
    return pl.BlockSpec(shape, lambda i: tuple(0 for _ in shape))


_NODE_OUT = [
    jax.ShapeDtypeStruct((N, C), jnp.float32),
    jax.ShapeDtypeStruct((N, 1), jnp.float32),
    jax.ShapeDtypeStruct((N, 1), jnp.float32),
]
_NODE_OUT_SPECS = [
    pl.BlockSpec((NB, C), lambda i: (i, 0)),
    pl.BlockSpec((NB, 1), lambda i: (i, 0)),
    pl.BlockSpec((NB, 1), lambda i: (i, 0)),
]


def _node_first(x, W, att_src, att_dst):
    return pl.pallas_call(
        _node_first_body,
        grid=(N // NB,),
        in_specs=[pl.BlockSpec((NB, x.shape[1]), lambda i: (i, 0)),
                  _rep(W.shape), _rep((1, C)), _rep((1, C))],
        out_specs=_NODE_OUT_SPECS,
        out_shape=_NODE_OUT,
    )(x, W, att_src.reshape(1, C), att_dst.reshape(1, C))


def _node_mid(acc, den, bias_prev, W, att_src, att_dst):
    return pl.pallas_call(
        _node_body,
        grid=(N // NB,),
        in_specs=[pl.BlockSpec((2, NB, C), lambda i: (0, i, 0)),
                  pl.BlockSpec((2, NB, 1), lambda i: (0, i, 0)),
                  _rep((1, C)), _rep(W.shape), _rep((1, C)), _rep((1, C))],
        out_specs=_NODE_OUT_SPECS,
        out_shape=_NODE_OUT,
    )(acc, den, bias_prev.reshape(1, C), W, att_src.reshape(1, C),
      att_dst.reshape(1, C))


def _ae_body(ea_ref, wcat_ref, attcat_ref, out_ref):
    prod = _bdot(ea_ref[...], wcat_ref[...]) * attcat_ref[...]
    for l in range(NLAYERS):
        out_ref[:, l] = prod[:, l * C:(l + 1) * C].sum(-1)


def _ae_all(edge_attr, conv_params):
    wcat = jnp.concatenate([p[1] for p in conv_params], axis=1)
    attcat = jnp.concatenate([p[4].reshape(1, C) for p in conv_params], axis=1)
    return pl.pallas_call(
        _ae_body,
        grid=(E // EB,),
        in_specs=[pl.BlockSpec((EB, edge_attr.shape[1]), lambda i: (i, 0)),
                  _rep(wcat.shape), _rep(attcat.shape)],
        out_specs=pl.BlockSpec((EB, 8), lambda i: (i, 0)),
        out_shape=jax.ShapeDtypeStruct((E, 8), jnp.float32),
    )(edge_attr, wcat, attcat)


def _mlp_body(acc_ref, den_ref, bias_ref, w1, b1, w2, b2, w3, b3, w4, b4,
              out_ref):
    dn = den_ref[0, :, 0] + den_ref[1, :, 0]
    inv = (1.0 / (dn + 1e-16))[:, None]
    h = _leaky((acc_ref[0] + acc_ref[1]) * inv + bias_ref[...])
    h = jnp.maximum(_bdot(h, w1[...]) + b1[...], 0.0)
    h = jnp.maximum(_bdot(h, w2[...]) + b2[...], 0.0)
    h = jnp.maximum(_bdot(h, w3[...]) + b3[...], 0.0)
    out_ref[...] = _bdot(h, w4[...]) + b4[...]


def _mlp_head(acc, den, bias_prev, lin_params):
    d_out = lin_params[-1][0].shape[1]
    args, specs = [], []
    for (w, b) in lin_params:
        args += [w, b.reshape(1, -1)]
        specs += [_rep(w.shape), _rep((1, b.shape[0]))]
    return pl.pallas_call(
        _mlp_body,
        grid=(N // NB,),
        in_specs=[pl.BlockSpec((2, NB, C), lambda i: (0, i, 0)),
                  pl.BlockSpec((2, NB, 1), lambda i: (0, i, 0)),
                  _rep((1, C))] + specs,
        out_specs=pl.BlockSpec((NB, d_out), lambda i: (i, 0)),
        out_shape=jax.ShapeDtypeStruct((N, d_out), jnp.float32),
    )(acc, den, bias_prev.reshape(1, C), *args)


def _edge_phase(hw, s, d, src, dst, ae):
    alpha = _leaky(s[src] + d[dst] + ae)
    ex = jnp.exp(jnp.minimum(alpha, 80.0))
    den = jax.ops.segment_sum(ex, dst, num_segments=N)
    acc = jax.ops.segment_sum(hw[src] * ex[:, None], dst, num_segments=N)
    acc2 = jnp.stack([acc, jnp.zeros_like(acc)])
    den2 = jnp.stack([den, jnp.zeros_like(den)])[..., None]
    return acc2, den2


def kernel(x, edge_index, edge_attr, conv_params, lin_params):
    src = edge_index[0]
    dst = edge_index[1]
    ae_all = _ae_all(edge_attr, conv_params)

    acc = den = None
    for li, (W, We, att_src, att_dst, att_edge, bias) in enumerate(conv_params):
        if li == 0:
            hw, s, d = _node_first(x, W, att_src, att_dst)
        else:
            bias_prev = conv_params[li - 1][5]
            hw, s, d = _node_mid(acc, den, bias_prev, W, att_src, att_dst)
        acc, den = _edge_phase(hw, s.reshape(-1), d.reshape(-1), src, dst,
                               ae_all[:, li])
    return _mlp_head(acc, den, conv_params[-1][5], lin_params)


# speedup vs baseline: 13.0909x; 13.0909x over previous
import functools

import jax
import jax.numpy as jnp
from jax.experimental import pallas as pl
from jax.experimental.pallas import tpu as pltpu
from jax.experimental.pallas import tpu_sc as plsc

N = 10000
E = 320000
C = 64
NLAYERS = 5
NB = 2000
EB = 2000


def _leaky(v):
    return jnp.where(v > 0, v, 0.2 * v)


def _bdot(a, b):
    return jax.lax.dot(a.astype(jnp.bfloat16), b.astype(jnp.bfloat16),
                       preferred_element_type=jnp.float32)


def _node_first_body(x_ref, w_ref, asrc_ref, adst_ref, hw_ref, s_ref, d_ref):
    hw = _bdot(x_ref[...], w_ref[...])
    hw_ref[...] = hw
    s_ref[...] = (hw * asrc_ref[...]).sum(-1)[:, None]
    d_ref[...] = (hw * adst_ref[...]).sum(-1)[:, None]


def _node_body(acc_ref, den_ref, bias_ref, w_ref, asrc_ref, adst_ref,
               hw_ref, s_ref, d_ref):
    dn = den_ref[0, :, 0] + den_ref[1, :, 0]
    inv = (1.0 / (dn + 1e-16))[:, None]
    act = _leaky((acc_ref[0] + acc_ref[1]) * inv + bias_ref[...])
    hw = _bdot(act, w_ref[...])
    hw_ref[...] = hw
    s_ref[...] = (hw * asrc_ref[...]).sum(-1)[:, None]
    d_ref[...] = (hw * adst_ref[...]).sum(-1)[:, None]


def _rep(shape):
    return pl.BlockSpec(shape, lambda i: tuple(0 for _ in shape))


_NODE_OUT = [
    jax.ShapeDtypeStruct((N, C), jnp.float32),
    jax.ShapeDtypeStruct((N, 1), jnp.float32),
    jax.ShapeDtypeStruct((N, 1), jnp.float32),
]
_NODE_OUT_SPECS = [
    pl.BlockSpec((NB, C), lambda i: (i, 0)),
    pl.BlockSpec((NB, 1), lambda i: (i, 0)),
    pl.BlockSpec((NB, 1), lambda i: (i, 0)),
]


def _node_first(x, W, att_src, att_dst):
    return pl.pallas_call(
        _node_first_body,
        grid=(N // NB,),
        in_specs=[pl.BlockSpec((NB, x.shape[1]), lambda i: (i, 0)),
                  _rep(W.shape), _rep((1, C)), _rep((1, C))],
        out_specs=_NODE_OUT_SPECS,
        out_shape=_NODE_OUT,
    )(x, W, att_src.reshape(1, C), att_dst.reshape(1, C))


def _node_mid(acc, den, bias_prev, W, att_src, att_dst):
    return pl.pallas_call(
        _node_body,
        grid=(N // NB,),
        in_specs=[pl.BlockSpec((2, NB, C), lambda i: (0, i, 0)),
                  pl.BlockSpec((2, NB, 1), lambda i: (0, i, 0)),
                  _rep((1, C)), _rep(W.shape), _rep((1, C)), _rep((1, C))],
        out_specs=_NODE_OUT_SPECS,
        out_shape=_NODE_OUT,
    )(acc, den, bias_prev.reshape(1, C), W, att_src.reshape(1, C),
      att_dst.reshape(1, C))


def _ae_body(ea_ref, wcat_ref, attcat_ref, out_ref):
    prod = _bdot(ea_ref[...], wcat_ref[...]) * attcat_ref[...]
    for l in range(NLAYERS):
        out_ref[:, l] = prod[:, l * C:(l + 1) * C].sum(-1)


def _ae_all(edge_attr, conv_params):
    wcat = jnp.concatenate([p[1] for p in conv_params], axis=1)
    attcat = jnp.concatenate([p[4].reshape(1, C) for p in conv_params], axis=1)
    return pl.pallas_call(
        _ae_body,
        grid=(E // EB,),
        in_specs=[pl.BlockSpec((EB, edge_attr.shape[1]), lambda i: (i, 0)),
                  _rep(wcat.shape), _rep(attcat.shape)],
        out_specs=pl.BlockSpec((EB, 8), lambda i: (i, 0)),
        out_shape=jax.ShapeDtypeStruct((E, 8), jnp.float32),
    )(edge_attr, wcat, attcat)


def _mlp_body(acc_ref, den_ref, bias_ref, w1, b1, w2, b2, w3, b3, w4, b4,
              out_ref):
    dn = den_ref[0, :, 0] + den_ref[1, :, 0]
    inv = (1.0 / (dn + 1e-16))[:, None]
    h = _leaky((acc_ref[0] + acc_ref[1]) * inv + bias_ref[...])
    h = jnp.maximum(_bdot(h, w1[...]) + b1[...], 0.0)
    h = jnp.maximum(_bdot(h, w2[...]) + b2[...], 0.0)
    h = jnp.maximum(_bdot(h, w3[...]) + b3[...], 0.0)
    out_ref[...] = _bdot(h, w4[...]) + b4[...]


def _mlp_head(acc, den, bias_prev, lin_params):
    d_out = lin_params[-1][0].shape[1]
    args, specs = [], []
    for (w, b) in lin_params:
        args += [w, b.reshape(1, -1)]
        specs += [_rep(w.shape), _rep((1, b.shape[0]))]
    return pl.pallas_call(
        _mlp_body,
        grid=(N // NB,),
        in_specs=[pl.BlockSpec((2, NB, C), lambda i: (0, i, 0)),
                  pl.BlockSpec((2, NB, 1), lambda i: (0, i, 0)),
                  _rep((1, C))] + specs,
        out_specs=pl.BlockSpec((NB, d_out), lambda i: (i, 0)),
        out_shape=jax.ShapeDtypeStruct((N, d_out), jnp.float32),
    )(acc, den, bias_prev.reshape(1, C), *args)


K = 400
NCH = 25
NROW = N // 16
NDEN = 1000


def _edge_sc_body(hw_hbm, s_hbm, d_hbm, src_hbm, dst_hbm, ae_hbm, zacc_hbm,
                  zden_hbm, acc_out, den0_out, den1_out, h_sh, acc_sh, den_sh,
                  s_t, d_t, src_t, dst_t, ae_t, ex_t, rows_t,
                  gsem, sem1, sem2):
    c = jax.lax.axis_index("c")
    w = jax.lax.axis_index("s")
    wid = c * 16 + w
    r0 = w * NDEN

    @pl.when(w < 10)
    def _():
        pltpu.sync_copy(hw_hbm.at[pl.ds(r0, NDEN)], h_sh.at[pl.ds(r0, NDEN)])
        pltpu.sync_copy(zacc_hbm.at[pl.ds(r0, NDEN)], acc_sh.at[pl.ds(r0, NDEN)])
        pltpu.sync_copy(zden_hbm.at[pl.ds(r0, NDEN)],
                        den_sh.at[pl.ds(r0, NDEN)])

    pltpu.sync_copy(s_hbm, s_t)
    pltpu.sync_copy(d_hbm, d_t)
    plsc.subcore_barrier()

    def chunk(j, carry):
        pltpu.sync_copy(src_hbm.at[wid, j], src_t)
        pltpu.sync_copy(dst_hbm.at[wid, j], dst_t)
        pltpu.sync_copy(ae_hbm.at[wid, j], ae_t)
        g = pltpu.async_copy(h_sh.at[src_t], rows_t, gsem)
        for v in range(K // 16):
            sl = pl.ds(v * 16, 16)
            a = (plsc.load_gather(s_t, [src_t[sl]])
                 + plsc.load_gather(d_t, [dst_t[sl]])
                 + ae_t[sl])
            a = jnp.where(a > 0, a, 0.2 * a)
            ex_t[sl] = jnp.exp(jnp.minimum(a, 80.0))
        g.wait()

        def scale(r, carry2):
            ev = plsc.load_gather(ex_t, [jnp.full((16,), r, jnp.int32)])
            for c4 in range(C // 16):
                csl = pl.ds(c4 * 16, 16)
                rows_t[r, csl] = rows_t[r, csl] * ev
            return carry2

        jax.lax.fori_loop(0, K, scale, 0)
        pltpu.async_copy(ex_t, den_sh.at[dst_t], sem1, add=True).wait()
        pltpu.async_copy(rows_t, acc_sh.at[dst_t], sem2, add=True).wait()
        return carry

    jax.lax.fori_loop(0, NCH, chunk, 0)
    plsc.subcore_barrier()

    @pl.when(w < 10)
    def _():
        pltpu.sync_copy(acc_sh.at[pl.ds(r0, NDEN)], acc_out.at[c, pl.ds(r0, NDEN)])

        @pl.when(c == 0)
        def _():
            pltpu.sync_copy(den_sh.at[pl.ds(r0, NDEN)], den0_out.at[pl.ds(r0, NDEN)])

        @pl.when(c == 1)
        def _():
            pltpu.sync_copy(den_sh.at[pl.ds(r0, NDEN)], den1_out.at[pl.ds(r0, NDEN)])


@functools.partial(jax.jit, static_argnames=())
def _edge_phase_sc(hw, s, d, srcR, dstR, aeR, zacc, zden):
    acc, den0, den1 = pl.kernel(
        _edge_sc_body,
        out_type=[jax.ShapeDtypeStruct((2, N, C), jnp.float32),
                  jax.ShapeDtypeStruct((N,), jnp.float32),
                  jax.ShapeDtypeStruct((N,), jnp.float32)],
        mesh=plsc.VectorSubcoreMesh(core_axis_name="c", subcore_axis_name="s"),
        compiler_params=pltpu.CompilerParams(use_tc_tiling_on_sc=False,
                                             needs_layout_passes=False),
        scratch_types=[
            pltpu.VMEM_SHARED((N, C), jnp.float32),
            pltpu.VMEM_SHARED((N, C), jnp.float32),
            pltpu.VMEM_SHARED((N,), jnp.float32),
            pltpu.VMEM((N,), jnp.float32),
            pltpu.VMEM((N,), jnp.float32),
            pltpu.VMEM((K,), jnp.int32),
            pltpu.VMEM((K,), jnp.int32),
            pltpu.VMEM((K,), jnp.float32),
            pltpu.VMEM((K,), jnp.float32),
            pltpu.VMEM((K, C), jnp.float32),
            pltpu.SemaphoreType.DMA,
            pltpu.SemaphoreType.DMA,
            pltpu.SemaphoreType.DMA,
        ],
    )(hw, s, d, srcR, dstR, aeR, zacc, zden)
    return acc, jnp.stack([den0, den1], 0)[..., None]


def kernel(x, edge_index, edge_attr, conv_params, lin_params):
    srcR = edge_index[0].reshape(32, NCH, K)
    dstR = edge_index[1].reshape(32, NCH, K)
    ae_all = _ae_all(edge_attr, conv_params)
    zacc = jnp.zeros((N, C), jnp.float32)
    zden = jnp.zeros((N,), jnp.float32)

    acc = den = None
    for li, (W, We, att_src, att_dst, att_edge, bias) in enumerate(conv_params):
        if li == 0:
            hw, s, d = _node_first(x, W, att_src, att_dst)
        else:
            bias_prev = conv_params[li - 1][5]
            hw, s, d = _node_mid(acc, den, bias_prev, W, att_src, att_dst)
        aeR = ae_all[:, li].reshape(32, NCH, K)
        acc, den = _edge_phase_sc(hw, s.reshape(-1), d.reshape(-1), srcR, dstR,
                                  aeR, zacc, zden)
    return _mlp_head(acc, den, conv_params[-1][5], lin_params)


# speedup vs baseline: 15.6523x; 1.1957x over previous
import functools

import jax
import jax.numpy as jnp
from jax.experimental import pallas as pl
from jax.experimental.pallas import tpu as pltpu
from jax.experimental.pallas import tpu_sc as plsc

N = 10000
E = 320000
C = 64
NLAYERS = 5
NB = 2000
EB = 8000


def _leaky(v):
    return jnp.where(v > 0, v, 0.2 * v)


def _bdot(a, b):
    return jax.lax.dot(a.astype(jnp.bfloat16), b.astype(jnp.bfloat16),
                       preferred_element_type=jnp.float32)


def _node_first_body(x_ref, w_ref, asrc_ref, adst_ref, hw_ref, s_ref, d_ref):
    hw = _bdot(x_ref[...], w_ref[...])
    hw_ref[...] = hw
    s_ref[...] = (hw * asrc_ref[...]).sum(-1)[:, None]
    d_ref[...] = (hw * adst_ref[...]).sum(-1)[:, None]


def _node_body(acc_ref, den_ref, bias_ref, w_ref, asrc_ref, adst_ref,
               hw_ref, s_ref, d_ref):
    dn = den_ref[0, :, 0] + den_ref[1, :, 0]
    inv = (1.0 / (dn + 1e-16))[:, None]
    act = _leaky((acc_ref[0] + acc_ref[1]) * inv + bias_ref[...])
    hw = _bdot(act, w_ref[...])
    hw_ref[...] = hw
    s_ref[...] = (hw * asrc_ref[...]).sum(-1)[:, None]
    d_ref[...] = (hw * adst_ref[...]).sum(-1)[:, None]


def _rep(shape):
    return pl.BlockSpec(shape, lambda i: tuple(0 for _ in shape))


_NODE_OUT = [
    jax.ShapeDtypeStruct((N, C), jnp.float32),
    jax.ShapeDtypeStruct((N, 1), jnp.float32),
    jax.ShapeDtypeStruct((N, 1), jnp.float32),
]
_NODE_OUT_SPECS = [
    pl.BlockSpec((NB, C), lambda i: (i, 0)),
    pl.BlockSpec((NB, 1), lambda i: (i, 0)),
    pl.BlockSpec((NB, 1), lambda i: (i, 0)),
]


def _node_first(x, W, att_src, att_dst):
    return pl.pallas_call(
        _node_first_body,
        grid=(N // NB,),
        in_specs=[pl.BlockSpec((NB, x.shape[1]), lambda i: (i, 0)),
                  _rep(W.shape), _rep((1, C)), _rep((1, C))],
        out_specs=_NODE_OUT_SPECS,
        out_shape=_NODE_OUT,
    )(x, W, att_src.reshape(1, C), att_dst.reshape(1, C))


def _node_mid(acc, den, bias_prev, W, att_src, att_dst):
    return pl.pallas_call(
        _node_body,
        grid=(N // NB,),
        in_specs=[pl.BlockSpec((2, NB, C), lambda i: (0, i, 0)),
                  pl.BlockSpec((2, NB, 1), lambda i: (0, i, 0)),
                  _rep((1, C)), _rep(W.shape), _rep((1, C)), _rep((1, C))],
        out_specs=_NODE_OUT_SPECS,
        out_shape=_NODE_OUT,
    )(acc, den, bias_prev.reshape(1, C), W, att_src.reshape(1, C),
      att_dst.reshape(1, C))


def _ae_body(ea_ref, wcat_ref, attcat_ref, gsel_ref, out_ref):
    prod = _bdot(ea_ref[...], wcat_ref[...]) * attcat_ref[...]
    out_ref[...] = jax.lax.dot(prod, gsel_ref[...],
                               precision=jax.lax.Precision.HIGHEST,
                               preferred_element_type=jnp.float32)


def _ae_all(edge_attr, conv_params):
    wcat = jnp.concatenate([p[1] for p in conv_params], axis=1)
    attcat = jnp.concatenate([p[4].reshape(1, C) for p in conv_params], axis=1)
    gsel = jnp.repeat(jnp.eye(NLAYERS, 8, dtype=jnp.float32), C, axis=0)
    return pl.pallas_call(
        _ae_body,
        grid=(E // EB,),
        in_specs=[pl.BlockSpec((EB, edge_attr.shape[1]), lambda i: (i, 0)),
                  _rep(wcat.shape), _rep(attcat.shape), _rep(gsel.shape)],
        out_specs=pl.BlockSpec((EB, 8), lambda i: (i, 0)),
        out_shape=jax.ShapeDtypeStruct((E, 8), jnp.float32),
    )(edge_attr, wcat, attcat, gsel)


def _mlp_body(acc_ref, den_ref, bias_ref, w1, b1, w2, b2, w3, b3, w4, b4,
              out_ref):
    dn = den_ref[0, :, 0] + den_ref[1, :, 0]
    inv = (1.0 / (dn + 1e-16))[:, None]
    h = _leaky((acc_ref[0] + acc_ref[1]) * inv + bias_ref[...])
    h = jnp.maximum(_bdot(h, w1[...]) + b1[...], 0.0)
    h = jnp.maximum(_bdot(h, w2[...]) + b2[...], 0.0)
    h = jnp.maximum(_bdot(h, w3[...]) + b3[...], 0.0)
    out_ref[...] = _bdot(h, w4[...]) + b4[...]


def _mlp_head(acc, den, bias_prev, lin_params):
    d_out = lin_params[-1][0].shape[1]
    args, specs = [], []
    for (w, b) in lin_params:
        args += [w, b.reshape(1, -1)]
        specs += [_rep(w.shape), _rep((1, b.shape[0]))]
    return pl.pallas_call(
        _mlp_body,
        grid=(N // NB,),
        in_specs=[pl.BlockSpec((2, NB, C), lambda i: (0, i, 0)),
                  pl.BlockSpec((2, NB, 1), lambda i: (0, i, 0)),
                  _rep((1, C))] + specs,
        out_specs=pl.BlockSpec((NB, d_out), lambda i: (i, 0)),
        out_shape=jax.ShapeDtypeStruct((N, d_out), jnp.float32),
    )(acc, den, bias_prev.reshape(1, C), *args)


K = 400
NCH = 25
NROW = N // 16
NDEN = 1000


def _edge_sc_body(hw_hbm, s_hbm, d_hbm, src_hbm, dst_hbm, ae_hbm, zacc_hbm,
                  zden_hbm, acc_out, den0_out, den1_out, h_sh, acc_sh, den_sh,
                  s_t, d_t, src_t, dst_t, ae_t, ex_t, rows_t,
                  gsem, sem1, sem2):
    c = jax.lax.axis_index("c")
    w = jax.lax.axis_index("s")
    wid = c * 16 + w
    r0 = w * NDEN

    @pl.when(w < 10)
    def _():
        pltpu.sync_copy(hw_hbm.at[pl.ds(r0, NDEN)], h_sh.at[pl.ds(r0, NDEN)])
        pltpu.sync_copy(zacc_hbm.at[pl.ds(r0, NDEN)], acc_sh.at[pl.ds(r0, NDEN)])
        pltpu.sync_copy(zden_hbm.at[pl.ds(r0, NDEN)],
                        den_sh.at[pl.ds(r0, NDEN)])

    pltpu.sync_copy(s_hbm, s_t)
    pltpu.sync_copy(d_hbm, d_t)
    plsc.subcore_barrier()

    def chunk(j, carry):
        ia = pltpu.async_copy(src_hbm.at[wid, j], src_t, sem1)
        ib = pltpu.async_copy(dst_hbm.at[wid, j], dst_t, sem1)
        ic = pltpu.async_copy(ae_hbm.at[wid, j], ae_t, sem1)
        ia.wait()
        g = pltpu.async_copy(h_sh.at[src_t], rows_t, gsem)
        ib.wait()
        ic.wait()
        for v in range(K // 16):
            sl = pl.ds(v * 16, 16)
            a = (plsc.load_gather(s_t, [src_t[sl]])
                 + plsc.load_gather(d_t, [dst_t[sl]])
                 + ae_t[sl])
            a = jnp.where(a > 0, a, 0.2 * a)
            ex_t[sl] = jnp.exp(jnp.minimum(a, 80.0))
        g.wait()

        def scale(r, carry2):
            ev = plsc.load_gather(ex_t, [jnp.full((16,), r, jnp.int32)])
            for c4 in range(C // 16):
                csl = pl.ds(c4 * 16, 16)
                rows_t[r, csl] = rows_t[r, csl] * ev
            return carry2

        jax.lax.fori_loop(0, K, scale, 0)
        sd = pltpu.async_copy(ex_t, den_sh.at[dst_t], sem2, add=True)
        sa = pltpu.async_copy(rows_t, acc_sh.at[dst_t], sem2, add=True)
        sd.wait()
        sa.wait()
        return carry

    jax.lax.fori_loop(0, NCH, chunk, 0)
    plsc.subcore_barrier()

    @pl.when(w < 10)
    def _():
        pltpu.sync_copy(acc_sh.at[pl.ds(r0, NDEN)], acc_out.at[c, pl.ds(r0, NDEN)])

        @pl.when(c == 0)
        def _():
            pltpu.sync_copy(den_sh.at[pl.ds(r0, NDEN)], den0_out.at[pl.ds(r0, NDEN)])

        @pl.when(c == 1)
        def _():
            pltpu.sync_copy(den_sh.at[pl.ds(r0, NDEN)], den1_out.at[pl.ds(r0, NDEN)])


@functools.partial(jax.jit, static_argnames=())
def _edge_phase_sc(hw, s, d, srcR, dstR, aeR, zacc, zden):
    acc, den0, den1 = pl.kernel(
        _edge_sc_body,
        out_type=[jax.ShapeDtypeStruct((2, N, C), jnp.float32),
                  jax.ShapeDtypeStruct((N,), jnp.float32),
                  jax.ShapeDtypeStruct((N,), jnp.float32)],
        mesh=plsc.VectorSubcoreMesh(core_axis_name="c", subcore_axis_name="s"),
        compiler_params=pltpu.CompilerParams(use_tc_tiling_on_sc=False,
                                             needs_layout_passes=False),
        scratch_types=[
            pltpu.VMEM_SHARED((N, C), jnp.float32),
            pltpu.VMEM_SHARED((N, C), jnp.float32),
            pltpu.VMEM_SHARED((N,), jnp.float32),
            pltpu.VMEM((N,), jnp.float32),
            pltpu.VMEM((N,), jnp.float32),
            pltpu.VMEM((K,), jnp.int32),
            pltpu.VMEM((K,), jnp.int32),
            pltpu.VMEM((K,), jnp.float32),
            pltpu.VMEM((K,), jnp.float32),
            pltpu.VMEM((K, C), jnp.float32),
            pltpu.SemaphoreType.DMA,
            pltpu.SemaphoreType.DMA,
            pltpu.SemaphoreType.DMA,
        ],
    )(hw, s, d, srcR, dstR, aeR, zacc, zden)
    return acc, jnp.stack([den0, den1], 0)[..., None]


def kernel(x, edge_index, edge_attr, conv_params, lin_params):
    srcR = edge_index[0].reshape(32, NCH, K)
    dstR = edge_index[1].reshape(32, NCH, K)
    ae_all = _ae_all(edge_attr, conv_params)
    zacc = jnp.zeros((N, C), jnp.float32)
    zden = jnp.zeros((N,), jnp.float32)

    acc = den = None
    for li, (W, We, att_src, att_dst, att_edge, bias) in enumerate(conv_params):
        if li == 0:
            hw, s, d = _node_first(x, W, att_src, att_dst)
        else:
            bias_prev = conv_params[li - 1][5]
            hw, s, d = _node_mid(acc, den, bias_prev, W, att_src, att_dst)
        aeR = ae_all[:, li].reshape(32, NCH, K)
        acc, den = _edge_phase_sc(hw, s.reshape(-1), d.reshape(-1), srcR, dstR,
                                  aeR, zacc, zden)
    return _mlp_head(acc, den, conv_params[-1][5], lin_params)


# speedup vs baseline: 16.0564x; 1.0258x over previous
import functools

import jax
import jax.numpy as jnp
from jax.experimental import pallas as pl
from jax.experimental.pallas import tpu as pltpu
from jax.experimental.pallas import tpu_sc as plsc

N = 10000
E = 320000
C = 64
NLAYERS = 5
NB = 2000
EB = 8000


def _leaky(v):
    return jnp.where(v > 0, v, 0.2 * v)


def _bdot(a, b):
    return jax.lax.dot(a.astype(jnp.bfloat16), b.astype(jnp.bfloat16),
                       preferred_element_type=jnp.float32)


def _node_first_body(x_ref, w_ref, asrc_ref, adst_ref, hw_ref, s_ref, d_ref):
    hw = _bdot(x_ref[...], w_ref[...])
    hw_ref[...] = hw
    s_ref[...] = (hw * asrc_ref[...]).sum(-1)[:, None]
    d_ref[...] = (hw * adst_ref[...]).sum(-1)[:, None]


def _node_body(acc_ref, den_ref, bias_ref, w_ref, asrc_ref, adst_ref,
               hw_ref, s_ref, d_ref):
    dn = den_ref[0, :, 0] + den_ref[1, :, 0]
    inv = (1.0 / (dn + 1e-16))[:, None]
    act = _leaky((acc_ref[0] + acc_ref[1]) * inv + bias_ref[...])
    hw = _bdot(act, w_ref[...])
    hw_ref[...] = hw
    s_ref[...] = (hw * asrc_ref[...]).sum(-1)[:, None]
    d_ref[...] = (hw * adst_ref[...]).sum(-1)[:, None]


def _rep(shape):
    return pl.BlockSpec(shape, lambda i: tuple(0 for _ in shape))


_NODE_OUT = [
    jax.ShapeDtypeStruct((N, C), jnp.float32),
    jax.ShapeDtypeStruct((N, 1), jnp.float32),
    jax.ShapeDtypeStruct((N, 1), jnp.float32),
]
_NODE_OUT_SPECS = [
    pl.BlockSpec((NB, C), lambda i: (i, 0)),
    pl.BlockSpec((NB, 1), lambda i: (i, 0)),
    pl.BlockSpec((NB, 1), lambda i: (i, 0)),
]


def _node_first(x, W, att_src, att_dst):
    return pl.pallas_call(
        _node_first_body,
        grid=(N // NB,),
        in_specs=[pl.BlockSpec((NB, x.shape[1]), lambda i: (i, 0)),
                  _rep(W.shape), _rep((1, C)), _rep((1, C))],
        out_specs=_NODE_OUT_SPECS,
        out_shape=_NODE_OUT,
    )(x, W, att_src.reshape(1, C), att_dst.reshape(1, C))


def _node_mid(acc, den, bias_prev, W, att_src, att_dst):
    return pl.pallas_call(
        _node_body,
        grid=(N // NB,),
        in_specs=[pl.BlockSpec((2, NB, C), lambda i: (0, i, 0)),
                  pl.BlockSpec((2, NB, 1), lambda i: (0, i, 0)),
                  _rep((1, C)), _rep(W.shape), _rep((1, C)), _rep((1, C))],
        out_specs=_NODE_OUT_SPECS,
        out_shape=_NODE_OUT,
    )(acc, den, bias_prev.reshape(1, C), W, att_src.reshape(1, C),
      att_dst.reshape(1, C))


def _ae_body(ea_ref, wcat_ref, attcat_ref, gsel_ref, out_ref):
    prod = _bdot(ea_ref[...], wcat_ref[...]) * attcat_ref[...]
    out_ref[...] = jax.lax.dot(prod, gsel_ref[...],
                               precision=jax.lax.Precision.HIGHEST,
                               preferred_element_type=jnp.float32)


def _ae_all(edge_attr, conv_params):
    wcat = jnp.concatenate([p[1] for p in conv_params], axis=1)
    attcat = jnp.concatenate([p[4].reshape(1, C) for p in conv_params], axis=1)
    gsel = jnp.repeat(jnp.eye(NLAYERS, 8, dtype=jnp.float32), C, axis=0)
    return pl.pallas_call(
        _ae_body,
        grid=(E // EB,),
        in_specs=[pl.BlockSpec((EB, edge_attr.shape[1]), lambda i: (i, 0)),
                  _rep(wcat.shape), _rep(attcat.shape), _rep(gsel.shape)],
        out_specs=pl.BlockSpec((EB, 8), lambda i: (i, 0)),
        out_shape=jax.ShapeDtypeStruct((E, 8), jnp.float32),
    )(edge_attr, wcat, attcat, gsel)


def _mlp_body(acc_ref, den_ref, bias_ref, w1, b1, w2, b2, w3, b3, w4, b4,
              out_ref):
    dn = den_ref[0, :, 0] + den_ref[1, :, 0]
    inv = (1.0 / (dn + 1e-16))[:, None]
    h = _leaky((acc_ref[0] + acc_ref[1]) * inv + bias_ref[...])
    h = jnp.maximum(_bdot(h, w1[...]) + b1[...], 0.0)
    h = jnp.maximum(_bdot(h, w2[...]) + b2[...], 0.0)
    h = jnp.maximum(_bdot(h, w3[...]) + b3[...], 0.0)
    out_ref[...] = _bdot(h, w4[...]) + b4[...]


def _mlp_head(acc, den, bias_prev, lin_params):
    d_out = lin_params[-1][0].shape[1]
    args, specs = [], []
    for (w, b) in lin_params:
        args += [w, b.reshape(1, -1)]
        specs += [_rep(w.shape), _rep((1, b.shape[0]))]
    return pl.pallas_call(
        _mlp_body,
        grid=(N // NB,),
        in_specs=[pl.BlockSpec((2, NB, C), lambda i: (0, i, 0)),
                  pl.BlockSpec((2, NB, 1), lambda i: (0, i, 0)),
                  _rep((1, C))] + specs,
        out_specs=pl.BlockSpec((NB, d_out), lambda i: (i, 0)),
        out_shape=jax.ShapeDtypeStruct((N, d_out), jnp.float32),
    )(acc, den, bias_prev.reshape(1, C), *args)


K = 400
NCH = 25
NROW = N // 16
NDEN = 1000


def _edge_sc_body(hw_hbm, s_hbm, d_hbm, src_hbm, dst_hbm, ae_hbm, zacc_hbm,
                  zden_hbm, acc_out, den0_out, den1_out, h_sh, acc_sh, den_sh,
                  s_t, d_t, src_t, dst_t, ae_t, ex_t, rows_t,
                  gsem, sem1, sem2):
    c = jax.lax.axis_index("c")
    w = jax.lax.axis_index("s")
    wid = c * 16 + w
    r0 = w * NDEN

    @pl.when(w < 10)
    def _():
        pltpu.sync_copy(hw_hbm.at[pl.ds(r0, NDEN)], h_sh.at[pl.ds(r0, NDEN)])
        pltpu.sync_copy(zacc_hbm.at[pl.ds(r0, NDEN)], acc_sh.at[pl.ds(r0, NDEN)])
        pltpu.sync_copy(zden_hbm.at[pl.ds(r0, NDEN)],
                        den_sh.at[pl.ds(r0, NDEN)])

    pltpu.sync_copy(s_hbm, s_t)
    pltpu.sync_copy(d_hbm, d_t)
    plsc.subcore_barrier()

    def chunk(j, carry):
        ia = pltpu.async_copy(src_hbm.at[wid, j], src_t, sem1)
        ib = pltpu.async_copy(dst_hbm.at[wid, j], dst_t, sem1)
        ic = pltpu.async_copy(ae_hbm.at[wid, j], ae_t, sem1)
        ia.wait()
        g = pltpu.async_copy(h_sh.at[src_t], rows_t, gsem)
        ib.wait()
        ic.wait()
        for v in range(K // 16):
            sl = pl.ds(v * 16, 16)
            a = (plsc.load_gather(s_t, [src_t[sl]])
                 + plsc.load_gather(d_t, [dst_t[sl]])
                 + ae_t[sl])
            a = jnp.where(a > 0, a, 0.2 * a)
            ex_t[sl] = jnp.exp(jnp.minimum(a, 80.0))
        g.wait()

        def scale(r8, carry2):
            base = r8 * 8
            for u in range(8):
                r = base + u
                ev = plsc.load_gather(ex_t, [jnp.full((16,), r, jnp.int32)])
                for c4 in range(C // 16):
                    csl = pl.ds(c4 * 16, 16)
                    rows_t[r, csl] = rows_t[r, csl] * ev
            return carry2

        jax.lax.fori_loop(0, K // 8, scale, 0)
        sd = pltpu.async_copy(ex_t, den_sh.at[dst_t], sem2, add=True)
        sa = pltpu.async_copy(rows_t, acc_sh.at[dst_t], sem2, add=True)
        sd.wait()
        sa.wait()
        return carry

    jax.lax.fori_loop(0, NCH, chunk, 0)
    plsc.subcore_barrier()

    @pl.when(w < 10)
    def _():
        pltpu.sync_copy(acc_sh.at[pl.ds(r0, NDEN)], acc_out.at[c, pl.ds(r0, NDEN)])

        @pl.when(c == 0)
        def _():
            pltpu.sync_copy(den_sh.at[pl.ds(r0, NDEN)], den0_out.at[pl.ds(r0, NDEN)])

        @pl.when(c == 1)
        def _():
            pltpu.sync_copy(den_sh.at[pl.ds(r0, NDEN)], den1_out.at[pl.ds(r0, NDEN)])


@functools.partial(jax.jit, static_argnames=())
def _edge_phase_sc(hw, s, d, srcR, dstR, aeR, zacc, zden):
    acc, den0, den1 = pl.kernel(
        _edge_sc_body,
        out_type=[jax.ShapeDtypeStruct((2, N, C), jnp.float32),
                  jax.ShapeDtypeStruct((N,), jnp.float32),
                  jax.ShapeDtypeStruct((N,), jnp.float32)],
        mesh=plsc.VectorSubcoreMesh(core_axis_name="c", subcore_axis_name="s"),
        compiler_params=pltpu.CompilerParams(use_tc_tiling_on_sc=False,
                                             needs_layout_passes=False),
        scratch_types=[
            pltpu.VMEM_SHARED((N, C), jnp.float32),
            pltpu.VMEM_SHARED((N, C), jnp.float32),
            pltpu.VMEM_SHARED((N,), jnp.float32),
            pltpu.VMEM((N,), jnp.float32),
            pltpu.VMEM((N,), jnp.float32),
            pltpu.VMEM((K,), jnp.int32),
            pltpu.VMEM((K,), jnp.int32),
            pltpu.VMEM((K,), jnp.float32),
            pltpu.VMEM((K,), jnp.float32),
            pltpu.VMEM((K, C), jnp.float32),
            pltpu.SemaphoreType.DMA,
            pltpu.SemaphoreType.DMA,
            pltpu.SemaphoreType.DMA,
        ],
    )(hw, s, d, srcR, dstR, aeR, zacc, zden)
    return acc, jnp.stack([den0, den1], 0)[..., None]


def kernel(x, edge_index, edge_attr, conv_params, lin_params):
    srcR = edge_index[0].reshape(32, NCH, K)
    dstR = edge_index[1].reshape(32, NCH, K)
    ae_all = _ae_all(edge_attr, conv_params)
    zacc = jnp.zeros((N, C), jnp.float32)
    zden = jnp.zeros((N,), jnp.float32)

    acc = den = None
    for li, (W, We, att_src, att_dst, att_edge, bias) in enumerate(conv_params):
        if li == 0:
            hw, s, d = _node_first(x, W, att_src, att_dst)
        else:
            bias_prev = conv_params[li - 1][5]
            hw, s, d = _node_mid(acc, den, bias_prev, W, att_src, att_dst)
        aeR = ae_all[:, li].reshape(32, NCH, K)
        acc, den = _edge_phase_sc(hw, s.reshape(-1), d.reshape(-1), srcR, dstR,
                                  aeR, zacc, zden)
    return _mlp_head(acc, den, conv_params[-1][5], lin_params)
